# Initial kernel scaffold; baseline (speedup 1.0000x reference)
import jax
import jax.numpy as jnp
from jax.experimental import pallas as pl


def kernel(inputs, router_w, router_b, fc1_w, fc1_b, fc2_w, fc2_b):
    raise NotImplementedError("write your pallas kernel here")


# speedup vs baseline: 3.4268x; 3.4268x over previous
import jax
import jax.numpy as jnp
from jax import lax
from jax.experimental import pallas as pl
from jax.experimental.pallas import tpu as pltpu

_BM = 256
_POS_CHUNK = 256


def _router_body(x_ref, rw_ref, rb_ref, mask_ref, pos_ref, gates_ref, cnt_ref):
    S, E = mask_ref.shape
    logits = jnp.dot(x_ref[...], rw_ref[...]) + rb_ref[...]

    iota_e = lax.broadcasted_iota(jnp.int32, (S, E), 1)
    big = jnp.int32(E)
    v1 = jnp.max(logits, axis=1, keepdims=True)
    idx1 = jnp.min(jnp.where(logits == v1, iota_e, big), axis=1, keepdims=True)
    oh1 = iota_e == idx1
    l2 = jnp.where(oh1, jnp.float32(-1e30), logits)
    v2 = jnp.max(l2, axis=1, keepdims=True)
    idx2 = jnp.min(jnp.where(l2 == v2, iota_e, big), axis=1, keepdims=True)
    oh2 = iota_e == idx2
    maskb = oh1 | oh2
    mask = maskb.astype(jnp.float32)

    denom = 1.0 + jnp.exp(v2 - v1)
    gates = jnp.where(maskb, jnp.exp(logits - v1) / denom, 0.0)

    mask_ref[...] = mask
    gates_ref[...] = gates
    cnt_ref[...] = jnp.sum(mask, axis=0, keepdims=True).astype(jnp.int32)

    for c in range(S // _POS_CHUNK):
        row_t = lax.broadcasted_iota(jnp.int32, (_POS_CHUNK, S), 0) + c * _POS_CHUNK
        col_t = lax.broadcasted_iota(jnp.int32, (_POS_CHUNK, S), 1)
        lt = (col_t < row_t).astype(jnp.float32)
        pos_ref[c * _POS_CHUNK:(c + 1) * _POS_CHUNK, :] = jnp.dot(
            lt, mask, preferred_element_type=jnp.float32)


def _ffn_body(x_ref, mask_ref, pos_ref, gates_ref, cnt_ref,
              fc1w_ref, fc1b_ref, fc2w_ref, fc2b_ref, out_ref):
    e = pl.program_id(0)
    j = pl.program_id(1)
    S, E = mask_ref.shape

    @pl.when((e == 0) & (j == 0))
    def _init():
        out_ref[...] = jnp.zeros_like(out_ref)

    cnt = cnt_ref[0, e]

    @pl.when(j * _BM < cnt)
    def _block():
        lane = lax.broadcasted_iota(jnp.int32, (S, E), 1)
        is_e = lane == e
        mcol = jnp.sum(jnp.where(is_e, mask_ref[...], 0.0), axis=1, keepdims=True)
        pcol = jnp.sum(jnp.where(is_e, pos_ref[...], 0.0), axis=1, keepdims=True)
        gcol = jnp.sum(jnp.where(is_e, gates_ref[...], 0.0), axis=1, keepdims=True)

        rid = (lax.broadcasted_iota(jnp.int32, (S, _BM), 1)
               + j * _BM).astype(jnp.float32)
        selT = jnp.where((pcol == rid) & (mcol > 0), 1.0, 0.0)

        xg = lax.dot_general(
            selT.astype(jnp.bfloat16), x_ref[...],
            (((0,), (0,)), ((), ())),
            preferred_element_type=jnp.float32)

        h = jnp.dot(xg.astype(jnp.bfloat16), fc1w_ref[0],
                    preferred_element_type=jnp.float32)
        h = h + fc1b_ref[0]
        h = 0.5 * h * (1.0 + lax.erf(h * jnp.float32(0.7071067811865476)))
        y = jnp.dot(h.astype(jnp.bfloat16), fc2w_ref[0],
                    preferred_element_type=jnp.float32)
        y = y + fc2b_ref[0]

        gsel = (selT * gcol).astype(jnp.bfloat16)
        out_ref[...] += jnp.dot(gsel, y.astype(jnp.bfloat16),
                                preferred_element_type=jnp.float32)


def kernel(inputs, router_w, router_b, fc1_w, fc1_b, fc2_w, fc2_b):
    B, S0, D = inputs.shape
    E = router_w.shape[1]
    H = fc1_w.shape[2]
    S = B * S0

    x = inputs.reshape(S, D)
    rb = router_b.reshape(1, E)

    mask, pos, gates, counts = pl.pallas_call(
        _router_body,
        out_shape=(
            jax.ShapeDtypeStruct((S, E), jnp.float32),
            jax.ShapeDtypeStruct((S, E), jnp.float32),
            jax.ShapeDtypeStruct((S, E), jnp.float32),
            jax.ShapeDtypeStruct((1, E), jnp.int32),
        ),
    )(x, router_w, rb)

    jmax = S // _BM
    out = pl.pallas_call(
        _ffn_body,
        grid=(E, jmax),
        in_specs=[
            pl.BlockSpec((S, D), lambda e, j: (0, 0)),
            pl.BlockSpec((S, E), lambda e, j: (0, 0)),
            pl.BlockSpec((S, E), lambda e, j: (0, 0)),
            pl.BlockSpec((S, E), lambda e, j: (0, 0)),
            pl.BlockSpec(memory_space=pltpu.SMEM),
            pl.BlockSpec((1, D, H), lambda e, j: (e, 0, 0)),
            pl.BlockSpec((1, 1, H), lambda e, j: (e, 0, 0)),
            pl.BlockSpec((1, H, D), lambda e, j: (e, 0, 0)),
            pl.BlockSpec((1, 1, D), lambda e, j: (e, 0, 0)),
        ],
        out_specs=pl.BlockSpec((S, D), lambda e, j: (0, 0)),
        out_shape=jax.ShapeDtypeStruct((S, D), jnp.float32),
        compiler_params=pltpu.CompilerParams(
            dimension_semantics=("arbitrary", "arbitrary")),
    )(
        x.astype(jnp.bfloat16),
        mask, pos, gates, counts,
        fc1_w.astype(jnp.bfloat16),
        fc1_b.reshape(E, 1, H),
        fc2_w.astype(jnp.bfloat16),
        fc2_b.reshape(E, 1, D),
    )
    return out.reshape(B, S0, D)
